# baseline (speedup 1.0000x reference)
import functools

import jax
import jax.numpy as jnp
from jax.experimental import pallas as pl

H = 224
W = 192
CIN = 64
NC = 16
C12 = 256
LB = 8
GL = 4
TA = LB * W
TB = GL * W
NTH = 2
TBH = TB // NTH

def _bdot(a, b, dims):
    return jax.lax.dot_general(a.astype(jnp.bfloat16), b.astype(jnp.bfloat16),
                               (dims, ((), ())),
                               preferred_element_type=jnp.float32)


def _b(a):
    return a.astype(jnp.bfloat16).astype(jnp.float32)


def _xdot(v, s, dims=((1,), (0,))):
    vh = v.astype(jnp.bfloat16)
    vl = (v - vh.astype(jnp.float32)).astype(jnp.bfloat16)
    sb = s.astype(jnp.bfloat16)
    acc = jax.lax.dot_general(vh, sb, (dims, ((), ())),
                              preferred_element_type=jnp.float32)
    return acc + jax.lax.dot_general(vl, sb, (dims, ((), ())),
                                     preferred_element_type=jnp.float32)


def _lrelu(x):
    return jnp.where(x >= 0, x, 0.01 * x)


def _argmax_lanes(z):
    m = jnp.max(z, axis=1, keepdims=True)
    ii = jax.lax.broadcasted_iota(jnp.int32, z.shape, 1)
    return jnp.min(jnp.where(z == m, ii, jnp.int32(2**30)), axis=1,
                   keepdims=True)


def _onehot(col, n):
    ii = jax.lax.broadcasted_iota(jnp.int32, (col.shape[0], n), 1)
    return (ii == col).astype(jnp.bfloat16)


def _stage12_body(x_ref, w0_ref, b0_ref, w1_ref, b1_ref, w2_ref, b2_ref,
                  cw0_ref, cb0_ref, bd1_ref, cb1_ref, bd2_ref, cb2_ref,
                  out_ref):
    for l in range(LB):
        X = x_ref[pl.ds(l * W, W), :]
        t = _bdot(X, w0_ref[l], ((1,), (1,))) + b0_ref[l][None, :]
        t = _lrelu(t)
        t = _bdot(t, w1_ref[l], ((1,), (1,))) + b1_ref[l][None, :]
        t = _lrelu(t)
        t = _bdot(t, w2_ref[l], ((1,), (1,))) + b2_ref[l][None, :]
        inds1 = _argmax_lanes(t)
        y0all = _bdot(X, cw0_ref[l], ((1,), (0,))) + cb0_ref[l][None, :]
        y1all = _bdot(_lrelu(y0all), bd1_ref[l], ((1,), (0,)))
        y1all = y1all + cb1_ref[l][None, :]
        y2all = _bdot(_lrelu(y1all), bd2_ref[l], ((1,), (0,)))
        y2all = y2all + cb2_ref[l][None, :]
        ii = jax.lax.broadcasted_iota(jnp.int32, (W, NC * 32), 1)
        msk = (ii // 32 == inds1)
        zm = jnp.where(msk, y2all, jnp.float32(-jnp.inf))
        idx = _argmax_lanes(zm)
        inds2 = idx - inds1 * 32
        i12 = inds1 * NC + inds2 - 8
        out_ref[pl.ds(l * W, W), :] = i12


def _stage3_body(x_ref, i12_ref, w0_ref, b0_ref, w1_ref, b1_ref,
                 w2_ref, b2_ref, r64_ref, s_ref, out_ref):
    X = x_ref[...]
    i12 = i12_ref[...]
    e12 = jnp.clip(i12, 0, C12 - 1)
    p256 = _onehot(e12, C12)
    z0 = _bdot(p256, b0_ref[0], ((1,), (0,)))
    for hf in range(2):
        a0 = _bdot(X[:, hf * 32:(hf + 1) * 32], r64_ref[:32, :1024],
                  ((1,), (0,)))
        wt0 = _bdot(p256, w0_ref[0][:, hf * 1024:(hf + 1) * 1024], ((1,), (0,)))
        z0 = z0 + _xdot(_b(a0) * _b(wt0), s_ref[:1024, :])
    y0 = _lrelu(z0)
    a1 = _bdot(y0, r64_ref[:32, :1024], ((1,), (0,)))
    wt1 = _bdot(p256, w1_ref[0], ((1,), (0,)))
    z1 = _xdot(_b(a1) * _b(wt1), s_ref[:1024, :])
    z1 = z1 + _bdot(p256, b1_ref[0], ((1,), (0,)))
    y1 = _lrelu(z1)
    a2 = _bdot(y1, r64_ref[:32, :1024], ((1,), (0,)))
    z2s = []
    for l2 in range(TBH // W):
        p_l = p256[l2 * W:(l2 + 1) * W, :]
        wt2 = _bdot(p_l, w2_ref[0][l2 * C12:(l2 + 1) * C12, :], ((1,), (0,)))
        z2 = _xdot(_b(a2[l2 * W:(l2 + 1) * W, :]) * _b(wt2), s_ref[:1024, :])
        z2 = z2 + _bdot(p_l, b2_ref[0][l2 * C12:(l2 + 1) * C12, :], ((1,), (0,)))
        z2s.append(z2)
    z2 = jnp.concatenate(z2s, axis=0)
    inds3 = _argmax_lanes(z2)
    out_ref[...] = jnp.clip(i12 * NC + inds3 - 8, 0, C12 * NC - 1)


def kernel(x_in, c1_w0, c1_b0, c1_w1, c1_b1, c1_w2, c1_b2,
           c2_w0, c2_b0, c2_w1, c2_b1, c2_w2, c2_b2,
           c3_w0, c3_b0, c3_w1, c3_b1, c3_w2, c3_b2):
    f32 = jnp.float32
    bf16 = jnp.bfloat16
    x_l = x_in.transpose(0, 2, 3, 1).reshape(H * W, CIN).astype(bf16)
    cw0 = c2_w0.astype(bf16).reshape(H, NC, 64, 32).transpose(0, 2, 1, 3).reshape(H, 64, NC * 32)
    cb0 = c2_b0.reshape(H, NC * 32)
    eye = jnp.eye(NC, dtype=jnp.bfloat16)
    w1r = c2_w1.reshape(H, NC, 32, 32).astype(jnp.bfloat16)
    bd1 = (w1r[:, :, :, None, :] * eye[None, :, None, :, None]
           ).reshape(H, NC * 32, NC * 32)
    w2r = c2_w2.reshape(H, NC, 32, 32).astype(jnp.bfloat16)
    bd2 = (w2r[:, :, :, None, :] * eye[None, :, None, :, None]
           ).reshape(H, NC * 32, NC * 32)
    cb1 = c2_b1.reshape(H, NC * 32)
    cb2 = c2_b2.reshape(H, NC * 32)
    NG = H // GL
    w0f = c3_w0.astype(bf16).reshape(NG, C12, 2048)
    b0f = c3_b0.reshape(NG, C12, 32)
    w1f = c3_w1.astype(bf16).reshape(NG, C12, 1024)
    b1f = c3_b1.reshape(NG, C12, 32)
    w2f = c3_w2.astype(bf16).reshape(NG * NTH, GL * C12 // NTH, 1024)
    b2f = c3_b2.reshape(NG * NTH, GL * C12 // NTH, 32)
    i2048 = jnp.arange(2048)
    r64 = (i2048[None, :] // 32 == jnp.arange(64)[:, None]).astype(bf16)
    r32 = r64[:32, :1024]
    s2048 = (i2048[:, None] % 32 == jnp.arange(32)[None, :]).astype(bf16)

    grid_a = H // LB
    i12 = pl.pallas_call(
        _stage12_body,
        grid=(grid_a,),
        in_specs=[
            pl.BlockSpec((TA, CIN), lambda i: (i, 0)),
            pl.BlockSpec((LB, 32, 64), lambda i: (i, 0, 0)),
            pl.BlockSpec((LB, 32), lambda i: (i, 0)),
            pl.BlockSpec((LB, 32, 32), lambda i: (i, 0, 0)),
            pl.BlockSpec((LB, 32), lambda i: (i, 0)),
            pl.BlockSpec((LB, 16, 32), lambda i: (i, 0, 0)),
            pl.BlockSpec((LB, 16), lambda i: (i, 0)),
            pl.BlockSpec((LB, 64, NC * 32), lambda i: (i, 0, 0)),
            pl.BlockSpec((LB, NC * 32), lambda i: (i, 0)),
            pl.BlockSpec((LB, NC * 32, NC * 32), lambda i: (i, 0, 0)),
            pl.BlockSpec((LB, NC * 32), lambda i: (i, 0)),
            pl.BlockSpec((LB, NC * 32, NC * 32), lambda i: (i, 0, 0)),
            pl.BlockSpec((LB, NC * 32), lambda i: (i, 0)),
        ],
        out_specs=pl.BlockSpec((TA, 1), lambda i: (i, 0)),
        out_shape=jax.ShapeDtypeStruct((H * W, 1), jnp.int32),
    )(x_l, c1_w0.astype(bf16), c1_b0, c1_w1.astype(bf16), c1_b1,
      c1_w2.astype(bf16), c1_b2,
      cw0, cb0, bd1, cb1, bd2, cb2)

    out = pl.pallas_call(
        _stage3_body,
        grid=(NG, NTH),
        in_specs=[
            pl.BlockSpec((TBH, CIN), lambda g, t: (g * NTH + t, 0)),
            pl.BlockSpec((TBH, 1), lambda g, t: (g * NTH + t, 0)),
            pl.BlockSpec((1, C12, 2048), lambda g, t: (g, 0, 0)),
            pl.BlockSpec((1, C12, 32), lambda g, t: (g, 0, 0)),
            pl.BlockSpec((1, C12, 1024), lambda g, t: (g, 0, 0)),
            pl.BlockSpec((1, C12, 32), lambda g, t: (g, 0, 0)),
            pl.BlockSpec((1, GL * C12 // NTH, 1024), lambda g, t: (g * NTH + t, 0, 0)),
            pl.BlockSpec((1, GL * C12 // NTH, 32), lambda g, t: (g * NTH + t, 0, 0)),
            pl.BlockSpec((64, 2048), lambda g, t: (0, 0)),
            pl.BlockSpec((2048, 32), lambda g, t: (0, 0)),
        ],
        out_specs=pl.BlockSpec((TBH, 1), lambda g, t: (g * NTH + t, 0)),
        out_shape=jax.ShapeDtypeStruct((H * W, 1), jnp.int32),
    )(x_l, i12, w0f, b0f, w1f, b1f, w2f, b2f, r64, s2048)

    return out.reshape(1, 1, H, W)


# speedup vs baseline: 1.1056x; 1.1056x over previous
import functools

import jax
import jax.numpy as jnp
from jax.experimental import pallas as pl

H = 224
W = 192
CIN = 64
NC = 16
C12 = 256
LB = 8
GL = 4
TA = LB * W
TB = GL * W
NTH = 2
TBH = TB // NTH

def _bdot(a, b, dims):
    return jax.lax.dot_general(a.astype(jnp.bfloat16), b.astype(jnp.bfloat16),
                               (dims, ((), ())),
                               preferred_element_type=jnp.float32)


def _b(a):
    return a.astype(jnp.bfloat16).astype(jnp.float32)


def _xdot(v, s, dims=((1,), (0,))):
    vh = v.astype(jnp.bfloat16)
    vl = (v - vh.astype(jnp.float32)).astype(jnp.bfloat16)
    sb = s.astype(jnp.bfloat16)
    acc = jax.lax.dot_general(vh, sb, (dims, ((), ())),
                              preferred_element_type=jnp.float32)
    return acc + jax.lax.dot_general(vl, sb, (dims, ((), ())),
                                     preferred_element_type=jnp.float32)


def _lrelu(x):
    return jnp.where(x >= 0, x, 0.01 * x)


def _argmax_lanes(z):
    m = jnp.max(z, axis=1, keepdims=True)
    ii = jax.lax.broadcasted_iota(jnp.int32, z.shape, 1)
    return jnp.min(jnp.where(z == m, ii, jnp.int32(2**30)), axis=1,
                   keepdims=True)


def _onehot(col, n):
    ii = jax.lax.broadcasted_iota(jnp.int32, (col.shape[0], n), 1)
    return (ii == col).astype(jnp.bfloat16)


def _stage12_body(x_ref, w0_ref, b0_ref, w1_ref, b1_ref, w2_ref, b2_ref,
                  cw0_ref, cb0_ref, bd1_ref, cb1_ref, bd2_ref, cb2_ref,
                  out_ref):
    for l in range(LB):
        X = x_ref[pl.ds(l * W, W), :]
        t = _bdot(X, w0_ref[l], ((1,), (1,))) + b0_ref[l][None, :]
        t = _lrelu(t)
        t = _bdot(t, w1_ref[l], ((1,), (1,))) + b1_ref[l][None, :]
        t = _lrelu(t)
        t = _bdot(t, w2_ref[l], ((1,), (1,))) + b2_ref[l][None, :]
        inds1 = _argmax_lanes(t)
        y0all = _bdot(X, cw0_ref[l], ((1,), (0,))) + cb0_ref[l][None, :]
        y1all = _bdot(_lrelu(y0all), bd1_ref[l], ((1,), (0,)))
        y1all = y1all + cb1_ref[l][None, :]
        y2all = _bdot(_lrelu(y1all), bd2_ref[l], ((1,), (0,)))
        y2all = y2all + cb2_ref[l][None, :]
        ii = jax.lax.broadcasted_iota(jnp.int32, (W, NC * 32), 1)
        msk = (ii // 32 == inds1)
        zm = jnp.where(msk, y2all, jnp.float32(-jnp.inf))
        idx = _argmax_lanes(zm)
        inds2 = idx - inds1 * 32
        i12 = inds1 * NC + inds2 - 8
        out_ref[pl.ds(l * W, W), :] = i12


def _stage3_body(x_ref, i12_ref, w0_ref, b0_ref, w1_ref, b1_ref,
                 w2_ref, b2_ref, r64_ref, s_ref, out_ref):
    X = x_ref[...]
    i12 = i12_ref[...]
    e12 = jnp.clip(i12, 0, C12 - 1)
    p256 = _onehot(e12, C12)
    z0 = _bdot(p256, b0_ref[0], ((1,), (0,)))
    for hf in range(2):
        a0 = _bdot(X[:, hf * 32:(hf + 1) * 32], r64_ref[:32, :1024],
                  ((1,), (0,)))
        wt0 = _bdot(p256, w0_ref[0][:, hf * 1024:(hf + 1) * 1024], ((1,), (0,)))
        z0 = z0 + _bdot(_b(a0) * _b(wt0), s_ref[:1024, :], ((1,), (0,)))
    y0 = _lrelu(z0)
    a1 = _bdot(y0, r64_ref[:32, :1024], ((1,), (0,)))
    wt1 = _bdot(p256, w1_ref[0], ((1,), (0,)))
    z1 = _bdot(_b(a1) * _b(wt1), s_ref[:1024, :], ((1,), (0,)))
    z1 = z1 + _bdot(p256, b1_ref[0], ((1,), (0,)))
    y1 = _lrelu(z1)
    a2 = _bdot(y1, r64_ref[:32, :1024], ((1,), (0,)))
    z2s = []
    for l2 in range(TBH // W):
        p_l = p256[l2 * W:(l2 + 1) * W, :]
        wt2 = _bdot(p_l, w2_ref[0][l2 * C12:(l2 + 1) * C12, :], ((1,), (0,)))
        z2 = _bdot(_b(a2[l2 * W:(l2 + 1) * W, :]) * _b(wt2), s_ref[:1024, :], ((1,), (0,)))
        z2 = z2 + _bdot(p_l, b2_ref[0][l2 * C12:(l2 + 1) * C12, :], ((1,), (0,)))
        z2s.append(z2)
    z2 = jnp.concatenate(z2s, axis=0)
    inds3 = _argmax_lanes(z2)
    out_ref[...] = jnp.clip(i12 * NC + inds3 - 8, 0, C12 * NC - 1)


def kernel(x_in, c1_w0, c1_b0, c1_w1, c1_b1, c1_w2, c1_b2,
           c2_w0, c2_b0, c2_w1, c2_b1, c2_w2, c2_b2,
           c3_w0, c3_b0, c3_w1, c3_b1, c3_w2, c3_b2):
    f32 = jnp.float32
    bf16 = jnp.bfloat16
    x_l = x_in.transpose(0, 2, 3, 1).reshape(H * W, CIN).astype(bf16)
    cw0 = c2_w0.astype(bf16).reshape(H, NC, 64, 32).transpose(0, 2, 1, 3).reshape(H, 64, NC * 32)
    cb0 = c2_b0.reshape(H, NC * 32)
    eye = jnp.eye(NC, dtype=jnp.bfloat16)
    w1r = c2_w1.reshape(H, NC, 32, 32).astype(jnp.bfloat16)
    bd1 = (w1r[:, :, :, None, :] * eye[None, :, None, :, None]
           ).reshape(H, NC * 32, NC * 32)
    w2r = c2_w2.reshape(H, NC, 32, 32).astype(jnp.bfloat16)
    bd2 = (w2r[:, :, :, None, :] * eye[None, :, None, :, None]
           ).reshape(H, NC * 32, NC * 32)
    cb1 = c2_b1.reshape(H, NC * 32)
    cb2 = c2_b2.reshape(H, NC * 32)
    NG = H // GL
    w0f = c3_w0.astype(bf16).reshape(NG, C12, 2048)
    b0f = c3_b0.reshape(NG, C12, 32)
    w1f = c3_w1.astype(bf16).reshape(NG, C12, 1024)
    b1f = c3_b1.reshape(NG, C12, 32)
    w2f = c3_w2.astype(bf16).reshape(NG * NTH, GL * C12 // NTH, 1024)
    b2f = c3_b2.reshape(NG * NTH, GL * C12 // NTH, 32)
    i2048 = jnp.arange(2048)
    r64 = (i2048[None, :] // 32 == jnp.arange(64)[:, None]).astype(bf16)
    r32 = r64[:32, :1024]
    s2048 = (i2048[:, None] % 32 == jnp.arange(32)[None, :]).astype(bf16)

    grid_a = H // LB
    i12 = pl.pallas_call(
        _stage12_body,
        grid=(grid_a,),
        in_specs=[
            pl.BlockSpec((TA, CIN), lambda i: (i, 0)),
            pl.BlockSpec((LB, 32, 64), lambda i: (i, 0, 0)),
            pl.BlockSpec((LB, 32), lambda i: (i, 0)),
            pl.BlockSpec((LB, 32, 32), lambda i: (i, 0, 0)),
            pl.BlockSpec((LB, 32), lambda i: (i, 0)),
            pl.BlockSpec((LB, 16, 32), lambda i: (i, 0, 0)),
            pl.BlockSpec((LB, 16), lambda i: (i, 0)),
            pl.BlockSpec((LB, 64, NC * 32), lambda i: (i, 0, 0)),
            pl.BlockSpec((LB, NC * 32), lambda i: (i, 0)),
            pl.BlockSpec((LB, NC * 32, NC * 32), lambda i: (i, 0, 0)),
            pl.BlockSpec((LB, NC * 32), lambda i: (i, 0)),
            pl.BlockSpec((LB, NC * 32, NC * 32), lambda i: (i, 0, 0)),
            pl.BlockSpec((LB, NC * 32), lambda i: (i, 0)),
        ],
        out_specs=pl.BlockSpec((TA, 1), lambda i: (i, 0)),
        out_shape=jax.ShapeDtypeStruct((H * W, 1), jnp.int32),
    )(x_l, c1_w0.astype(bf16), c1_b0, c1_w1.astype(bf16), c1_b1,
      c1_w2.astype(bf16), c1_b2,
      cw0, cb0, bd1, cb1, bd2, cb2)

    out = pl.pallas_call(
        _stage3_body,
        grid=(NG, NTH),
        in_specs=[
            pl.BlockSpec((TBH, CIN), lambda g, t: (g * NTH + t, 0)),
            pl.BlockSpec((TBH, 1), lambda g, t: (g * NTH + t, 0)),
            pl.BlockSpec((1, C12, 2048), lambda g, t: (g, 0, 0)),
            pl.BlockSpec((1, C12, 32), lambda g, t: (g, 0, 0)),
            pl.BlockSpec((1, C12, 1024), lambda g, t: (g, 0, 0)),
            pl.BlockSpec((1, C12, 32), lambda g, t: (g, 0, 0)),
            pl.BlockSpec((1, GL * C12 // NTH, 1024), lambda g, t: (g * NTH + t, 0, 0)),
            pl.BlockSpec((1, GL * C12 // NTH, 32), lambda g, t: (g * NTH + t, 0, 0)),
            pl.BlockSpec((64, 2048), lambda g, t: (0, 0)),
            pl.BlockSpec((2048, 32), lambda g, t: (0, 0)),
        ],
        out_specs=pl.BlockSpec((TBH, 1), lambda g, t: (g * NTH + t, 0)),
        out_shape=jax.ShapeDtypeStruct((H * W, 1), jnp.int32),
    )(x_l, i12, w0f, b0f, w1f, b1f, w2f, b2f, r64, s2048)

    return out.reshape(1, 1, H, W)
